# Initial kernel scaffold; baseline (speedup 1.0000x reference)
import jax
import jax.numpy as jnp
from jax.experimental import pallas as pl


def kernel(x, W):
    raise NotImplementedError("write your pallas kernel here")


# speedup vs baseline: 1.7192x; 1.7192x over previous
import functools

import jax
import jax.numpy as jnp
from jax import lax
from jax.experimental import pallas as pl
from jax.experimental.pallas import tpu as pltpu
from jax.experimental.pallas import tpu_sc as plsc

F = 26
V = 40000
B = 16384
L = 16


def _gather_body(w_hbm, xt_hbm, out_hbm, tab_v, idx_v, part_v):
    cid = lax.axis_index("c")
    sid = lax.axis_index("s")
    f = cid * 16 + sid

    @pl.when(f < F)
    def _():
        pltpu.sync_copy(w_hbm.at[pl.ds(f * V, V)], tab_v)
        pltpu.sync_copy(xt_hbm.at[f], idx_v)

        def body(i, carry):
            idx = idx_v[pl.ds(i * L, L)]
            part_v[pl.ds(i * L, L)] = plsc.load_gather(tab_v, [idx])
            return carry

        lax.fori_loop(0, B // L, body, 0)
        pltpu.sync_copy(part_v, out_hbm.at[f])


_sc_gather = functools.partial(
    pl.kernel,
    out_type=jax.ShapeDtypeStruct((F, B), jnp.float32),
    mesh=plsc.VectorSubcoreMesh(core_axis_name="c", subcore_axis_name="s"),
    compiler_params=pltpu.CompilerParams(needs_layout_passes=False),
    scratch_types=[
        pltpu.VMEM((V,), jnp.float32),
        pltpu.VMEM((B,), jnp.int32),
        pltpu.VMEM((B,), jnp.float32),
    ],
)(_gather_body)


def _reduce_body(p_ref, o_ref):
    o_ref[...] = jnp.sum(p_ref[...], axis=0, keepdims=True)


_tc_reduce = pl.pallas_call(
    _reduce_body,
    out_shape=jax.ShapeDtypeStruct((1, B), jnp.float32),
)


@jax.jit
def kernel(x, W):
    xt = x.T
    w_flat = W.reshape(-1)
    partials = _sc_gather(w_flat, xt)
    out = _tc_reduce(partials)
    return out.reshape(B, 1)
